# baseline (speedup 1.0000x reference)
import functools

import numpy as np
import jax
import jax.numpy as jnp
from jax.experimental import pallas as pl
from jax.experimental.pallas import tpu as pltpu


def _fused_net_kernel(x_ref, a1_ref, bp1_ref, b2_ref, bp2_ref,
                      fw_ref, fb_ref, o_ref, *, tb):
    f32 = jnp.float32
    bf16 = jnp.bfloat16
    tg = tb // 8

    x4 = x_ref[...].reshape(tg, 8, 28, 28)
    xi = x4.transpose(0, 2, 1, 3).reshape(tg, 28 * 8, 28).astype(bf16)

    lhs1 = jnp.concatenate(
        [jnp.pad(xi[:, 8 * ki:8 * (ki + 24), :], ((0, 0), (0, 0), (0, 4)))
         for ki in range(5)], axis=2).reshape(24 * tb, 160)
    acc1 = jnp.dot(lhs1, a1_ref[...], preferred_element_type=f32)

    hw = jnp.maximum(acc1[:, :128], acc1[:, 128:])
    hh = hw.reshape(tg, 12, 2, 8, 128)
    p = jnp.maximum(hh[:, :, 0], hh[:, :, 1])
    p1 = jnp.maximum(p + bp1_ref[...], 0.0).astype(bf16)
    p1 = p1.reshape(tg, 12 * 8, 128)

    lhs2 = jnp.concatenate(
        [p1[:, 8 * ki:8 * (ki + 8), :] for ki in range(5)],
        axis=2).reshape(8 * tb, 640)
    acc2 = jnp.dot(lhs2, b2_ref[...], preferred_element_type=f32)

    qw = jnp.maximum(acc2[:, :128], acc2[:, 128:])
    qh = qw.reshape(tg, 4, 2, 8, 128)
    qq = jnp.maximum(qh[:, :, 0], qh[:, :, 1])
    q = jnp.maximum(qq + bp2_ref[...], 0.0).astype(bf16)

    qc = jnp.concatenate([q[:, r] for r in range(4)], axis=2).reshape(tb, 512)
    out = jnp.dot(qc, fw_ref[...], preferred_element_type=f32) + fb_ref[...]
    o_ref[...] = out.astype(o_ref.dtype)


def _prep_params(w1, b1, w2, b2, fw1, fb1, fw2, fb2):
    f32 = jnp.float32
    bf16 = jnp.bfloat16
    w1 = w1.astype(f32)
    w2 = w2.astype(f32)
    fw1 = fw1.astype(f32)
    fw2 = fw2.astype(f32)

    kj, w, pw, owp = np.meshgrid(np.arange(5), np.arange(28), np.arange(2),
                                 np.arange(12), indexing="ij")
    m1 = jnp.asarray((w == 2 * owp + pw + kj).astype(np.float32))
    a1 = jnp.einsum("cik,kwpq->iwpqc", w1[:, 0], m1)
    a1 = jnp.pad(a1.reshape(5, 28, 2, 120),
                 ((0, 0), (0, 4), (0, 0), (0, 8))).reshape(160, 256)

    bp1 = jnp.pad(jnp.tile(b1.astype(f32), 12), (0, 8)).reshape(1, 128)

    kj, iw, pw2, owp2 = np.meshgrid(np.arange(5), np.arange(12), np.arange(2),
                                    np.arange(4), indexing="ij")
    m2 = jnp.asarray((iw == 2 * owp2 + pw2 + kj).astype(np.float32))
    b2m = jnp.einsum("dcik,kwpq->iwcpqd", w2, m2)
    b2m = jnp.pad(b2m.reshape(5, 120, 2, 80),
                  ((0, 0), (0, 8), (0, 0), (0, 48))).reshape(640, 256)

    bp2 = jnp.pad(jnp.tile(b2.astype(f32), 4), (0, 48)).reshape(1, 128)

    w1fc = fw1.reshape(50, 20, 4, 4)
    w1fc = jnp.transpose(w1fc, (2, 3, 1, 0)).reshape(4, 80, 50)
    fw1m = jnp.pad(w1fc, ((0, 0), (0, 48), (0, 0))).reshape(512, 50)
    fw = jnp.dot(fw1m, fw2.T)
    fb = (jnp.dot(fb1.astype(f32).reshape(1, 50), fw2.T)
          + fb2.astype(f32).reshape(1, 10))

    return a1.astype(bf16), bp1, b2m.astype(bf16), bp2, fw.astype(bf16), fb


def _net_pallas(xr, a1, bp1, b2m, bp2, fw, fb, *, tb):
    Bp = xr.shape[0]
    G = Bp // tb

    flops = G * 2 * (24 * tb * 160 * 256 + 8 * tb * 640 * 256 + tb * 512 * 10)
    bytes_accessed = (4 * xr.size + 2 * (a1.size + b2m.size + fw.size)
                      + 4 * (bp1.size + bp2.size + fb.size + Bp * 10))

    const2 = lambda g: (0, 0)
    return pl.pallas_call(
        functools.partial(_fused_net_kernel, tb=tb),
        out_shape=jax.ShapeDtypeStruct((Bp, 10), jnp.float32),
        grid=(G,),
        in_specs=[
            pl.BlockSpec((tb, 28, 28), lambda g: (g, 0, 0)),
            pl.BlockSpec((160, 256), const2),
            pl.BlockSpec((1, 128), const2),
            pl.BlockSpec((640, 256), const2),
            pl.BlockSpec((1, 128), const2),
            pl.BlockSpec((512, 10), const2),
            pl.BlockSpec((1, 10), const2),
        ],
        out_specs=pl.BlockSpec((tb, 10), lambda g: (g, 0)),
        compiler_params=pltpu.CompilerParams(dimension_semantics=("parallel",)),
        cost_estimate=pl.CostEstimate(flops=flops, transcendentals=0,
                                      bytes_accessed=bytes_accessed),
    )(xr, a1, bp1, b2m, bp2, fw, fb)


def kernel(x, w1, b1, w2, b2, fw1, fb1, fw2, fb2, *, tile_b=256):
    B = x.shape[0]
    assert x.shape[1:] == (1, 28, 28)
    tb = tile_b
    G = (B + tb - 1) // tb
    Bp = G * tb

    a1, bp1, b2m, bp2, fw, fb = _prep_params(w1, b1, w2, b2, fw1, fb1, fw2, fb2)

    nchunk = 2 if (B % (2 * tb) == 0) else 1
    cb = B // nchunk
    outs = []
    for c in range(nchunk):
        xc = x[c * cb:(c + 1) * cb].reshape(cb, 28, 28)
        Bc = ((cb + tb - 1) // tb) * tb
        if Bc != cb:
            xc = jnp.concatenate([xc, jnp.zeros((Bc - cb, 28, 28), xc.dtype)],
                                 axis=0)
        outs.append(_net_pallas(xc, a1, bp1, b2m, bp2, fw, fb, tb=tb)[:cb])
    out = outs[0] if nchunk == 1 else jnp.concatenate(outs, axis=0)
    return out[:B]


# speedup vs baseline: 1.7805x; 1.7805x over previous
import functools

import numpy as np
import jax
import jax.numpy as jnp
from jax.experimental import pallas as pl
from jax.experimental.pallas import tpu as pltpu


def _fused_net_kernel(x_ref, a1_ref, bp1_ref, b2_ref, bp2_ref,
                      fw_ref, fb_ref, o_ref, *, tb):
    f32 = jnp.float32
    bf16 = jnp.bfloat16
    tg = tb // 8

    x4 = x_ref[...].reshape(tg, 8, 28, 28)
    xi = x4.transpose(0, 2, 1, 3).reshape(tg, 28 * 8, 28).astype(bf16)

    lhs1 = jnp.concatenate(
        [jnp.pad(xi[:, 8 * ki:8 * (ki + 24), :], ((0, 0), (0, 0), (0, 4)))
         for ki in range(5)], axis=2).reshape(24 * tb, 160)
    acc1 = jnp.dot(lhs1, a1_ref[...], preferred_element_type=f32)

    hw = jnp.maximum(acc1[:, :128], acc1[:, 128:])
    hh = hw.reshape(tg, 12, 2, 8, 128)
    p = jnp.maximum(hh[:, :, 0], hh[:, :, 1])
    p1 = jnp.maximum(p + bp1_ref[...], 0.0).astype(bf16)
    p1 = p1.reshape(tg, 12 * 8, 128)

    lhs2 = jnp.concatenate(
        [p1[:, 8 * ki:8 * (ki + 8), :] for ki in range(5)],
        axis=2).reshape(8 * tb, 640)
    acc2 = jnp.dot(lhs2, b2_ref[...], preferred_element_type=f32)

    qw = jnp.maximum(acc2[:, :128], acc2[:, 128:])
    qh = qw.reshape(tg, 4, 2, 8, 128)
    qq = jnp.maximum(qh[:, :, 0], qh[:, :, 1])
    q = jnp.maximum(qq + bp2_ref[...], 0.0).astype(bf16)

    qc = jnp.concatenate([q[:, r] for r in range(4)], axis=2).reshape(tb, 512)
    out = jnp.dot(qc, fw_ref[...], preferred_element_type=f32) + fb_ref[...]
    o_ref[...] = out.astype(o_ref.dtype)


def _prep_params(w1, b1, w2, b2, fw1, fb1, fw2, fb2):
    f32 = jnp.float32
    bf16 = jnp.bfloat16
    w1 = w1.astype(f32)
    w2 = w2.astype(f32)
    fw1 = fw1.astype(f32)
    fw2 = fw2.astype(f32)

    kj, w, pw, owp = np.meshgrid(np.arange(5), np.arange(28), np.arange(2),
                                 np.arange(12), indexing="ij")
    m1 = jnp.asarray((w == 2 * owp + pw + kj).astype(np.float32))
    a1 = jnp.einsum("cik,kwpq->iwpqc", w1[:, 0], m1)
    a1 = jnp.pad(a1.reshape(5, 28, 2, 120),
                 ((0, 0), (0, 4), (0, 0), (0, 8))).reshape(160, 256)

    bp1 = jnp.pad(jnp.tile(b1.astype(f32), 12), (0, 8)).reshape(1, 128)

    kj, iw, pw2, owp2 = np.meshgrid(np.arange(5), np.arange(12), np.arange(2),
                                    np.arange(4), indexing="ij")
    m2 = jnp.asarray((iw == 2 * owp2 + pw2 + kj).astype(np.float32))
    b2m = jnp.einsum("dcik,kwpq->iwcpqd", w2, m2)
    b2m = jnp.pad(b2m.reshape(5, 120, 2, 80),
                  ((0, 0), (0, 8), (0, 0), (0, 48))).reshape(640, 256)

    bp2 = jnp.pad(jnp.tile(b2.astype(f32), 4), (0, 48)).reshape(1, 128)

    w1fc = fw1.reshape(50, 20, 4, 4)
    w1fc = jnp.transpose(w1fc, (2, 3, 1, 0)).reshape(4, 80, 50)
    fw1m = jnp.pad(w1fc, ((0, 0), (0, 48), (0, 0))).reshape(512, 50)
    fw = jnp.dot(fw1m, fw2.T)
    fb = (jnp.dot(fb1.astype(f32).reshape(1, 50), fw2.T)
          + fb2.astype(f32).reshape(1, 10))

    return a1.astype(bf16), bp1, b2m.astype(bf16), bp2, fw.astype(bf16), fb


def _net_pallas(xr, a1, bp1, b2m, bp2, fw, fb, *, tb):
    Bp = xr.shape[0]
    G = Bp // tb

    flops = G * 2 * (24 * tb * 160 * 256 + 8 * tb * 640 * 256 + tb * 512 * 10)
    bytes_accessed = (4 * xr.size + 2 * (a1.size + b2m.size + fw.size)
                      + 4 * (bp1.size + bp2.size + fb.size + Bp * 10))

    const2 = lambda g: (0, 0)
    return pl.pallas_call(
        functools.partial(_fused_net_kernel, tb=tb),
        out_shape=jax.ShapeDtypeStruct((Bp, 10), jnp.float32),
        grid=(G,),
        in_specs=[
            pl.BlockSpec((tb, 28, 28), lambda g: (g, 0, 0)),
            pl.BlockSpec((160, 256), const2),
            pl.BlockSpec((1, 128), const2),
            pl.BlockSpec((640, 256), const2),
            pl.BlockSpec((1, 128), const2),
            pl.BlockSpec((512, 10), const2),
            pl.BlockSpec((1, 10), const2),
        ],
        out_specs=pl.BlockSpec((tb, 10), lambda g: (g, 0)),
        compiler_params=pltpu.CompilerParams(dimension_semantics=("parallel",)),
        cost_estimate=pl.CostEstimate(flops=flops, transcendentals=0,
                                      bytes_accessed=bytes_accessed),
    )(xr, a1, bp1, b2m, bp2, fw, fb)


def kernel(x, w1, b1, w2, b2, fw1, fb1, fw2, fb2, *, tile_b=512):
    B = x.shape[0]
    assert x.shape[1:] == (1, 28, 28)
    tb = tile_b
    G = (B + tb - 1) // tb
    Bp = G * tb

    xr = x.reshape(B, 28, 28)
    if Bp != B:
        xr = jnp.concatenate([xr, jnp.zeros((Bp - B, 28, 28), xr.dtype)], axis=0)

    a1, bp1, b2m, bp2, fw, fb = _prep_params(w1, b1, w2, b2, fw1, fb1, fw2, fb2)

    out = _net_pallas(xr, a1, bp1, b2m, bp2, fw, fb, tb=tb)
    return out[:B]
